# baseline (speedup 1.0000x reference)
import functools

import jax
import jax.numpy as jnp
from jax import lax
from jax.experimental import pallas as pl
from jax.experimental.pallas import tpu as pltpu
from jax.experimental.pallas import tpu_sc as plsc

N = 50000
E = 800000
IN_DIM = 128
HID = 64
HALF = 32
NG = 64
NC = 8

NP = 50048
EP = 802816
EROWS = EP // 128
ACC_N = 50048
TRASH = 50000

f32 = jnp.float32


def _sc_deg_body(colp_hbm, out_hbm, acc, colbuf, ones_v, zero_v, stage_v, sem):
    c = lax.axis_index("c")
    s = lax.axis_index("s")
    wid = s * 2 + c

    @pl.loop(0, 8)
    def _fill(j):
        ones_v[pl.ds(j * 16, 16)] = jnp.ones((16,), f32)
        zero_v[pl.ds(j * 16, 16)] = jnp.zeros((16,), f32)

    @pl.loop(0, 25)
    def _zero(k):
        ck = s + 16 * k

        @pl.when(ck < 391)
        def _():
            pltpu.sync_copy(zero_v, acc.at[pl.ds(ck * 128, 128)])

    plsc.subcore_barrier()

    @pl.loop(0, 49)
    def _outer(i):
        rb = wid * 196 + i * 4
        pltpu.sync_copy(colp_hbm.at[pl.ds(rb, 4)], colbuf)
        for r in range(4):
            pltpu.sync_copy(ones_v, acc.at[colbuf.at[r]], add=True)

    plsc.subcore_barrier()

    pltpu.sync_copy(acc.at[pl.ds(s * 3128, 3128)], stage_v)
    pltpu.sync_copy(stage_v, out_hbm.at[pl.ds(c * NP + s * 3128, 3128)])


def _sc_agg_body(rowp_hbm, colp_hbm, hp_hbm, out_hbm, acc,
                 rowA, colA, rowB, colB, rowsA, rowsB, zero_v,
                 siA, siB, sgA, sgB, ssA, ssB):
    c = lax.axis_index("c")
    s = lax.axis_index("s")

    @pl.loop(0, 64)
    def _fillz(r):
        zero_v[r, pl.ds(0, 16)] = jnp.zeros((16,), f32)
        zero_v[r, pl.ds(16, 16)] = jnp.zeros((16,), f32)

    @pl.loop(0, 49)
    def _zero(k):
        ck = s + 16 * k

        @pl.when(ck < 782)
        def _():
            pltpu.sync_copy(zero_v, acc.at[pl.ds(ck * 64, 64)])

    plsc.subcore_barrier()

    def fire_idx(g, rowb, colb, sem):
        rb = s * 392 + g * 2
        pltpu.async_copy(rowp_hbm.at[pl.ds(rb, 2)], rowb, sem)
        pltpu.async_copy(colp_hbm.at[pl.ds(rb, 2)], colb, sem)

    def wait_idx(rowb, colb, sem):
        pltpu.make_async_copy(rowp_hbm.at[pl.ds(0, 2)], rowb, sem).wait()
        pltpu.make_async_copy(colp_hbm.at[pl.ds(0, 2)], colb, sem).wait()

    def fire_g(rowb, rowsb, sem):
        for r in range(2):
            pltpu.async_copy(hp_hbm.at[c].at[rowb.at[r]],
                             rowsb.at[pl.ds(r * 128, 128)], sem)

    def wait_g(rowb, rowsb, sem):
        for r in range(2):
            pltpu.make_async_copy(hp_hbm.at[c].at[rowb.at[r]],
                                  rowsb.at[pl.ds(r * 128, 128)], sem).wait()

    def fire_s(colb, rowsb, sem):
        for r in range(2):
            pltpu.async_copy(rowsb.at[pl.ds(r * 128, 128)],
                             acc.at[colb.at[r]], sem, add=True)

    def wait_s(colb, rowsb, sem):
        for r in range(2):
            pltpu.make_async_copy(rowsb.at[pl.ds(r * 128, 128)],
                                  acc.at[colb.at[r]], sem).wait()

    fire_idx(0, rowA, colA, siA)

    @pl.loop(0, 98)
    def _outer(i):
        @pl.when(i > 0)
        def _wsa():
            wait_s(colA, rowsA, ssA)

        @pl.when(i > 0)
        def _fia():
            fire_idx(2 * i, rowA, colA, siA)

        @pl.when(i > 0)
        def _sb():
            wait_g(rowB, rowsB, sgB)
            fire_s(colB, rowsB, ssB)

        wait_idx(rowA, colA, siA)
        fire_g(rowA, rowsA, sgA)

        @pl.when(i > 0)
        def _wsb():
            wait_s(colB, rowsB, ssB)

        fire_idx(2 * i + 1, rowB, colB, siB)
        wait_idx(rowB, colB, siB)
        fire_g(rowB, rowsB, sgB)

        wait_g(rowA, rowsA, sgA)
        fire_s(colA, rowsA, ssA)

    wait_g(rowB, rowsB, sgB)
    fire_s(colB, rowsB, ssB)
    wait_s(colA, rowsA, ssA)
    wait_s(colB, rowsB, ssB)

    plsc.subcore_barrier()

    @pl.loop(0, 17)
    def _out(k):
        base = s * 3128 + k * 184
        pltpu.sync_copy(acc.at[pl.ds(base, 184)],
                        rowsA.at[pl.ds(0, 184)])
        pltpu.sync_copy(rowsA.at[pl.ds(0, 184)],
                        out_hbm.at[c, pl.ds(base, 184)])


_SC_MESH = plsc.VectorSubcoreMesh(core_axis_name="c", subcore_axis_name="s")


def _build_sc_deg():
    return pl.kernel(
        _sc_deg_body,
        out_type=jax.ShapeDtypeStruct((2 * NP,), f32),
        mesh=_SC_MESH,
        scratch_types=[
            pltpu.VMEM_SHARED((ACC_N,), f32),
            pltpu.VMEM((4, 128), jnp.int32),
            pltpu.VMEM((128,), f32),
            pltpu.VMEM((128,), f32),
            pltpu.VMEM((3128,), f32),
            pltpu.SemaphoreType.DMA,
        ],
    )


def _build_sc_agg():
    return pl.kernel(
        _sc_agg_body,
        out_type=jax.ShapeDtypeStruct((2, NP, HALF), f32),
        mesh=_SC_MESH,
        scratch_types=[
            pltpu.VMEM_SHARED((ACC_N, HALF), f32),
            pltpu.VMEM((2, 128), jnp.int32),
            pltpu.VMEM((2, 128), jnp.int32),
            pltpu.VMEM((2, 128), jnp.int32),
            pltpu.VMEM((2, 128), jnp.int32),
            pltpu.VMEM((256, HALF), f32),
            pltpu.VMEM((256, HALF), f32),
            pltpu.VMEM((64, HALF), f32),
            pltpu.SemaphoreType.DMA,
            pltpu.SemaphoreType.DMA,
            pltpu.SemaphoreType.DMA,
            pltpu.SemaphoreType.DMA,
            pltpu.SemaphoreType.DMA,
            pltpu.SemaphoreType.DMA,
        ],
        compiler_params=pltpu.CompilerParams(use_tc_tiling_on_sc=False),
    )


_BLK = 1000
_NBLK = N // _BLK

_HBM = pl.BlockSpec(memory_space=pltpu.MemorySpace.HBM)


def _dinv_block(degp_hbm, d0, d1, i, sem):
    pltpu.async_copy(degp_hbm.at[0, pl.ds(i * _BLK, _BLK)], d0, sem)
    pltpu.async_copy(degp_hbm.at[1, pl.ds(i * _BLK, _BLK)], d1, sem).wait()
    pltpu.make_async_copy(degp_hbm.at[0, pl.ds(0, _BLK)], d0, sem).wait()
    return lax.rsqrt(d0[...] + d1[...] + 1.0)


def _load_pair(src_hbm, b0, b1, i, sem):
    pltpu.async_copy(src_hbm.at[0, pl.ds(i * _BLK, _BLK)], b0, sem)
    pltpu.async_copy(src_hbm.at[1, pl.ds(i * _BLK, _BLK)], b1, sem).wait()
    pltpu.make_async_copy(src_hbm.at[0, pl.ds(0, _BLK)], b0, sem).wait()
    return jnp.concatenate([b0[...], b1[...]], axis=1)


def _store_pair(x, dst_hbm, b0, b1, i, sem):
    b0[...] = x[:, :HALF]
    b1[...] = x[:, HALF:]
    pltpu.async_copy(b0, dst_hbm.at[0, pl.ds(i * _BLK, _BLK)], sem)
    pltpu.async_copy(b1, dst_hbm.at[1, pl.ds(i * _BLK, _BLK)], sem).wait()
    pltpu.make_async_copy(b0, dst_hbm.at[0, pl.ds(0, _BLK)], sem).wait()


def _tc1_body(x_ref, w_ref, degp_hbm, hp_hbm,
              d0, d1, o0, o1, sem):
    i = pl.program_id(0)
    h = jnp.dot(x_ref[...], w_ref[...], preferred_element_type=f32)
    dinv = _dinv_block(degp_hbm, d0, d1, i, sem)
    _store_pair(h * dinv, hp_hbm, o0, o1, i, sem)


def _tc_mid_body(agg_hbm, hpp_hbm, degp_hbm, b_ref, w_ref, hp_hbm,
                 d0, d1, a0, a1, p0, p1, o0, o1, sem):
    i = pl.program_id(0)
    a = _load_pair(agg_hbm, a0, a1, i, sem)
    hpv = _load_pair(hpp_hbm, p0, p1, i, sem)
    dinv = _dinv_block(degp_hbm, d0, d1, i, sem)
    s = jax.nn.relu(dinv * (a + hpv) + b_ref[...])
    h = jnp.dot(s, w_ref[...], preferred_element_type=f32)
    _store_pair(h * dinv, hp_hbm, o0, o1, i, sem)


def _tc_final_body(agg_hbm, hpp_hbm, degp_hbm, b_ref, batch_ref, wc_ref,
                   bc_ref, out_ref, d0, d1, a0, a1, p0, p1, psum, cnt, sem):
    i = pl.program_id(0)
    a = _load_pair(agg_hbm, a0, a1, i, sem)
    hpv = _load_pair(hpp_hbm, p0, p1, i, sem)
    dinv = _dinv_block(degp_hbm, d0, d1, i, sem)
    h = jax.nn.relu(dinv * (a + hpv) + b_ref[...])
    onehot_t = (lax.broadcasted_iota(jnp.int32, (NG, _BLK), 0)
                == batch_ref[0]).astype(f32)
    ps = jnp.dot(onehot_t, h, preferred_element_type=f32)
    ct = jnp.sum(onehot_t, axis=1, keepdims=True)

    @pl.when(i == 0)
    def _init():
        psum[...] = ps
        cnt[...] = ct

    @pl.when(i > 0)
    def _acc():
        psum[...] += ps
        cnt[...] += ct

    @pl.when(i == _NBLK - 1)
    def _fin():
        pooled = psum[...] / jnp.maximum(cnt[...], 1.0)
        out_ref[...] = (jnp.dot(pooled, wc_ref[...],
                                preferred_element_type=f32) + bc_ref[...])


def _make_tc1():
    return pl.pallas_call(
        _tc1_body,
        grid=(_NBLK,),
        in_specs=[
            pl.BlockSpec((_BLK, IN_DIM), lambda i: (i, 0)),
            pl.BlockSpec((IN_DIM, HID), lambda i: (0, 0)),
            _HBM,
        ],
        out_specs=_HBM,
        out_shape=jax.ShapeDtypeStruct((2, NP, HALF), f32),
        scratch_shapes=[
            pltpu.VMEM((_BLK, 1), f32),
            pltpu.VMEM((_BLK, 1), f32),
            pltpu.VMEM((_BLK, HALF), f32),
            pltpu.VMEM((_BLK, HALF), f32),
            pltpu.SemaphoreType.DMA,
        ],
    )


def _make_tc_mid():
    return pl.pallas_call(
        _tc_mid_body,
        grid=(_NBLK,),
        in_specs=[
            _HBM,
            _HBM,
            _HBM,
            pl.BlockSpec((1, HID), lambda i: (0, 0)),
            pl.BlockSpec((HID, HID), lambda i: (0, 0)),
        ],
        out_specs=_HBM,
        out_shape=jax.ShapeDtypeStruct((2, NP, HALF), f32),
        scratch_shapes=[
            pltpu.VMEM((_BLK, 1), f32),
            pltpu.VMEM((_BLK, 1), f32),
            pltpu.VMEM((_BLK, HALF), f32),
            pltpu.VMEM((_BLK, HALF), f32),
            pltpu.VMEM((_BLK, HALF), f32),
            pltpu.VMEM((_BLK, HALF), f32),
            pltpu.VMEM((_BLK, HALF), f32),
            pltpu.VMEM((_BLK, HALF), f32),
            pltpu.SemaphoreType.DMA,
        ],
    )


def _make_tc_final():
    return pl.pallas_call(
        _tc_final_body,
        grid=(_NBLK,),
        in_specs=[
            _HBM,
            _HBM,
            _HBM,
            pl.BlockSpec((1, HID), lambda i: (0, 0)),
            pl.BlockSpec((1, 1, _BLK), lambda i: (i, 0, 0)),
            pl.BlockSpec((HID, NC), lambda i: (0, 0)),
            pl.BlockSpec((1, NC), lambda i: (0, 0)),
        ],
        out_specs=pl.BlockSpec((NG, NC), lambda i: (0, 0)),
        out_shape=jax.ShapeDtypeStruct((NG, NC), f32),
        scratch_shapes=[
            pltpu.VMEM((_BLK, 1), f32),
            pltpu.VMEM((_BLK, 1), f32),
            pltpu.VMEM((_BLK, HALF), f32),
            pltpu.VMEM((_BLK, HALF), f32),
            pltpu.VMEM((_BLK, HALF), f32),
            pltpu.VMEM((_BLK, HALF), f32),
            pltpu.VMEM((NG, HID), f32),
            pltpu.VMEM((NG, 1), f32),
            pltpu.SemaphoreType.DMA,
        ],
    )


@jax.jit
def kernel(x, edge_index, batch, W1, b1, W2, b2, W3, b3, Wc, bc):
    sc_deg = _build_sc_deg()
    sc_agg = _build_sc_agg()
    tc1 = _make_tc1()
    tc_mid = _make_tc_mid()
    tc_final = _make_tc_final()

    pad = EP - E
    rowp = jnp.concatenate(
        [edge_index[0], jnp.zeros((pad,), jnp.int32)]).reshape(EROWS, 128)
    colp = jnp.concatenate(
        [edge_index[1], jnp.full((pad,), TRASH, jnp.int32)]).reshape(EROWS, 128)
    batch3 = batch.reshape(_NBLK, 1, _BLK)

    degp = sc_deg(colp).reshape(2, NP, 1)
    hp1 = tc1(x, W1, degp)
    agg1 = sc_agg(rowp, colp, hp1)
    hp2 = tc_mid(agg1, hp1, degp, b1.reshape(1, HID), W2)
    agg2 = sc_agg(rowp, colp, hp2)
    hp3 = tc_mid(agg2, hp2, degp, b2.reshape(1, HID), W3)
    agg3 = sc_agg(rowp, colp, hp3)
    return tc_final(agg3, hp3, degp, b3.reshape(1, HID), batch3,
                    Wc, bc.reshape(1, NC))


# speedup vs baseline: 1.2094x; 1.2094x over previous
import functools

import jax
import jax.numpy as jnp
from jax import lax
from jax.experimental import pallas as pl
from jax.experimental.pallas import tpu as pltpu
from jax.experimental.pallas import tpu_sc as plsc

N = 50000
E = 800000
IN_DIM = 128
HID = 64
HALF = 32
NG = 64
NC = 8

NP = 50048
EP = 802816
EROWS = EP // 128
ACC_N = 50048
TRASH = 50000

f32 = jnp.float32


def _sc_deg_body(colp_hbm, out_hbm, acc, colbuf, ones_v, zero_v, stage_v, sem):
    c = lax.axis_index("c")
    s = lax.axis_index("s")
    wid = s * 2 + c

    @pl.loop(0, 8)
    def _fill(j):
        ones_v[pl.ds(j * 16, 16)] = jnp.ones((16,), f32)
        zero_v[pl.ds(j * 16, 16)] = jnp.zeros((16,), f32)

    @pl.loop(0, 25)
    def _zero(k):
        ck = s + 16 * k

        @pl.when(ck < 391)
        def _():
            pltpu.sync_copy(zero_v, acc.at[pl.ds(ck * 128, 128)])

    plsc.subcore_barrier()

    @pl.loop(0, 49)
    def _outer(i):
        rb = wid * 196 + i * 4
        pltpu.sync_copy(colp_hbm.at[pl.ds(rb, 4)], colbuf)
        for r in range(4):
            pltpu.sync_copy(ones_v, acc.at[colbuf.at[r]], add=True)

    plsc.subcore_barrier()

    pltpu.sync_copy(acc.at[pl.ds(s * 3128, 3128)], stage_v)
    pltpu.sync_copy(stage_v, out_hbm.at[pl.ds(c * NP + s * 3128, 3128)])


def _sc_agg_body(rowp_hbm, colp_hbm, hp_hbm, out_hbm, acc,
                 row0, col0, row1, col1, row2, col2, row3, col3,
                 rows0, rows1, rows2, rows3, zero_v, stage_v,
                 si0, si1, si2, si3, sg0, sg1, sg2, sg3,
                 ss0, ss1, ss2, ss3):
    c = lax.axis_index("c")
    s = lax.axis_index("s")
    rowb = [row0, row1, row2, row3]
    colb = [col0, col1, col2, col3]
    rows = [rows0, rows1, rows2, rows3]
    si = [si0, si1, si2, si3]
    sg = [sg0, sg1, sg2, sg3]
    ss = [ss0, ss1, ss2, ss3]

    @pl.loop(0, 64)
    def _fillz(r):
        zero_v[r, pl.ds(0, 16)] = jnp.zeros((16,), f32)
        zero_v[r, pl.ds(16, 16)] = jnp.zeros((16,), f32)

    @pl.loop(0, 49)
    def _zero(k):
        ck = s + 16 * k

        @pl.when(ck < 782)
        def _():
            pltpu.sync_copy(zero_v, acc.at[pl.ds(ck * 64, 64)])

    plsc.subcore_barrier()

    def fire_idx(g, k):
        rb = s * 392 + g
        pltpu.async_copy(rowp_hbm.at[pl.ds(rb, 1)], rowb[k], si[k])
        pltpu.async_copy(colp_hbm.at[pl.ds(rb, 1)], colb[k], si[k])

    def wait_idx(k):
        pltpu.make_async_copy(rowp_hbm.at[pl.ds(0, 1)], rowb[k], si[k]).wait()
        pltpu.make_async_copy(colp_hbm.at[pl.ds(0, 1)], colb[k], si[k]).wait()

    def fire_g(k):
        pltpu.async_copy(hp_hbm.at[c].at[rowb[k].at[0]], rows[k], sg[k])

    def wait_g(k):
        pltpu.make_async_copy(hp_hbm.at[c].at[rowb[k].at[0]],
                              rows[k], sg[k]).wait()

    def fire_s(k):
        pltpu.async_copy(rows[k], acc.at[colb[k].at[0]], ss[k], add=True)

    def wait_s(k):
        pltpu.make_async_copy(rows[k], acc.at[colb[k].at[0]], ss[k]).wait()

    @pl.loop(0, 98)
    def _outer(i):
        for k in range(4):
            @pl.when(i > 0)
            def _fs(k=k):
                wait_g(k)
                fire_s(k)
        for k in range(4):
            @pl.when(i > 0)
            def _ws(k=k):
                wait_s(k)
            fire_idx(4 * i + k, k)
            wait_idx(k)
            fire_g(k)

    for k in range(4):
        wait_g(k)
        fire_s(k)
    for k in range(4):
        wait_s(k)

    plsc.subcore_barrier()

    @pl.loop(0, 17)
    def _out(k):
        base = s * 3128 + k * 184
        pltpu.sync_copy(acc.at[pl.ds(base, 184)], stage_v)
        pltpu.sync_copy(stage_v, out_hbm.at[c, pl.ds(base, 184)])


_SC_MESH = plsc.VectorSubcoreMesh(core_axis_name="c", subcore_axis_name="s")


def _build_sc_deg():
    return pl.kernel(
        _sc_deg_body,
        out_type=jax.ShapeDtypeStruct((2 * NP,), f32),
        mesh=_SC_MESH,
        scratch_types=[
            pltpu.VMEM_SHARED((ACC_N,), f32),
            pltpu.VMEM((4, 128), jnp.int32),
            pltpu.VMEM((128,), f32),
            pltpu.VMEM((128,), f32),
            pltpu.VMEM((3128,), f32),
            pltpu.SemaphoreType.DMA,
        ],
    )


def _build_sc_agg():
    return pl.kernel(
        _sc_agg_body,
        out_type=jax.ShapeDtypeStruct((2, NP, HALF), f32),
        mesh=_SC_MESH,
        scratch_types=(
            [pltpu.VMEM_SHARED((ACC_N, HALF), f32)]
            + [pltpu.VMEM((1, 128), jnp.int32) for _ in range(8)]
            + [pltpu.VMEM((128, HALF), f32) for _ in range(4)]
            + [pltpu.VMEM((64, HALF), f32), pltpu.VMEM((184, HALF), f32)]
            + [pltpu.SemaphoreType.DMA for _ in range(12)]
        ),
        compiler_params=pltpu.CompilerParams(use_tc_tiling_on_sc=False),
    )


_BLK = 1000
_NBLK = N // _BLK


def _tc1_body(x_ref, w_ref, degp_ref, hp_ref, dinv_ref):
    h = jnp.dot(x_ref[...], w_ref[...], preferred_element_type=f32)
    dtot = degp_ref[0] + degp_ref[1] + 1.0
    dinv = lax.rsqrt(dtot)
    dinv_ref[...] = dinv
    hp = h * dinv
    hp_ref[0] = hp[:, :HALF]
    hp_ref[1] = hp[:, HALF:]


def _tc_mid_body(agg_ref, hpp_ref, dinv_ref, b_ref, w_ref, hp_ref):
    a = jnp.concatenate([agg_ref[0], agg_ref[1]], axis=1)
    hpv = jnp.concatenate([hpp_ref[0], hpp_ref[1]], axis=1)
    s = jax.nn.relu(dinv_ref[...] * (a + hpv) + b_ref[...])
    h = jnp.dot(s, w_ref[...], preferred_element_type=f32)
    hp = h * dinv_ref[...]
    hp_ref[0] = hp[:, :HALF]
    hp_ref[1] = hp[:, HALF:]


def _tc_final_body(agg_ref, hpp_ref, dinv_ref, b_ref, batch_ref, wc_ref,
                   bc_ref, out_ref, psum, cnt):
    i = pl.program_id(0)
    a = jnp.concatenate([agg_ref[0], agg_ref[1]], axis=1)
    hpv = jnp.concatenate([hpp_ref[0], hpp_ref[1]], axis=1)
    h = jax.nn.relu(dinv_ref[...] * (a + hpv) + b_ref[...])
    onehot_t = (lax.broadcasted_iota(jnp.int32, (NG, _BLK), 0)
                == batch_ref[0]).astype(f32)
    ps = jnp.dot(onehot_t, h, preferred_element_type=f32)
    ct = jnp.sum(onehot_t, axis=1, keepdims=True)

    @pl.when(i == 0)
    def _init():
        psum[...] = ps
        cnt[...] = ct

    @pl.when(i > 0)
    def _acc():
        psum[...] += ps
        cnt[...] += ct

    @pl.when(i == _NBLK - 1)
    def _fin():
        pooled = psum[...] / jnp.maximum(cnt[...], 1.0)
        out_ref[...] = (jnp.dot(pooled, wc_ref[...],
                                preferred_element_type=f32) + bc_ref[...])


def _make_tc1():
    return pl.pallas_call(
        _tc1_body,
        grid=(_NBLK,),
        in_specs=[
            pl.BlockSpec((_BLK, IN_DIM), lambda i: (i, 0)),
            pl.BlockSpec((IN_DIM, HID), lambda i: (0, 0)),
            pl.BlockSpec((2, _BLK, 1), lambda i: (0, i, 0)),
        ],
        out_specs=[
            pl.BlockSpec((2, _BLK, HALF), lambda i: (0, i, 0)),
            pl.BlockSpec((_BLK, 1), lambda i: (i, 0)),
        ],
        out_shape=[
            jax.ShapeDtypeStruct((2, NP, HALF), f32),
            jax.ShapeDtypeStruct((N, 1), f32),
        ],
    )


def _make_tc_mid():
    return pl.pallas_call(
        _tc_mid_body,
        grid=(_NBLK,),
        in_specs=[
            pl.BlockSpec((2, _BLK, HALF), lambda i: (0, i, 0)),
            pl.BlockSpec((2, _BLK, HALF), lambda i: (0, i, 0)),
            pl.BlockSpec((_BLK, 1), lambda i: (i, 0)),
            pl.BlockSpec((1, HID), lambda i: (0, 0)),
            pl.BlockSpec((HID, HID), lambda i: (0, 0)),
        ],
        out_specs=[
            pl.BlockSpec((2, _BLK, HALF), lambda i: (0, i, 0)),
        ],
        out_shape=[jax.ShapeDtypeStruct((2, NP, HALF), f32)],
    )


def _make_tc_final():
    return pl.pallas_call(
        _tc_final_body,
        grid=(_NBLK,),
        in_specs=[
            pl.BlockSpec((2, _BLK, HALF), lambda i: (0, i, 0)),
            pl.BlockSpec((2, _BLK, HALF), lambda i: (0, i, 0)),
            pl.BlockSpec((_BLK, 1), lambda i: (i, 0)),
            pl.BlockSpec((1, HID), lambda i: (0, 0)),
            pl.BlockSpec((1, 1, _BLK), lambda i: (i, 0, 0)),
            pl.BlockSpec((HID, NC), lambda i: (0, 0)),
            pl.BlockSpec((1, NC), lambda i: (0, 0)),
        ],
        out_specs=pl.BlockSpec((NG, NC), lambda i: (0, 0)),
        out_shape=jax.ShapeDtypeStruct((NG, NC), f32),
        scratch_shapes=[
            pltpu.VMEM((NG, HID), f32),
            pltpu.VMEM((NG, 1), f32),
        ],
    )


@jax.jit
def kernel(x, edge_index, batch, W1, b1, W2, b2, W3, b3, Wc, bc):
    sc_deg = _build_sc_deg()
    sc_agg = _build_sc_agg()
    tc1 = _make_tc1()
    tc_mid = _make_tc_mid()
    tc_final = _make_tc_final()

    pad = EP - E
    rowp = jnp.concatenate(
        [edge_index[0], jnp.zeros((pad,), jnp.int32)]).reshape(EROWS, 128)
    colp = jnp.concatenate(
        [edge_index[1], jnp.full((pad,), TRASH, jnp.int32)]).reshape(EROWS, 128)
    batch3 = batch.reshape(_NBLK, 1, _BLK)

    degp = sc_deg(colp).reshape(2, NP, 1)
    hp1, dinv = tc1(x, W1, degp)
    agg1 = sc_agg(rowp, colp, hp1)
    hp2, = tc_mid(agg1, hp1, dinv, b1.reshape(1, HID), W2)
    agg2 = sc_agg(rowp, colp, hp2)
    hp3, = tc_mid(agg2, hp2, dinv, b2.reshape(1, HID), W3)
    agg3 = sc_agg(rowp, colp, hp3)
    return tc_final(agg3, hp3, dinv, b3.reshape(1, HID), batch3,
                    Wc, bc.reshape(1, NC))


# speedup vs baseline: 1.5741x; 1.3015x over previous
import functools

import jax
import jax.numpy as jnp
from jax import lax
from jax.experimental import pallas as pl
from jax.experimental.pallas import tpu as pltpu
from jax.experimental.pallas import tpu_sc as plsc

N = 50000
E = 800000
IN_DIM = 128
HID = 64
HALF = 32
NG = 64
NC = 8

NP = 50048
EP = 802816
EROWS = EP // 128
ACC_N = 50048
TRASH = 50000

f32 = jnp.float32


def _sc_deg_body(colp_hbm, out_hbm, acc, colbuf, ones_v, zero_v, stage_v, sem):
    c = lax.axis_index("c")
    s = lax.axis_index("s")
    wid = s * 2 + c

    @pl.loop(0, 8)
    def _fill(j):
        ones_v[pl.ds(j * 16, 16)] = jnp.ones((16,), f32)
        zero_v[pl.ds(j * 16, 16)] = jnp.zeros((16,), f32)

    @pl.loop(0, 25)
    def _zero(k):
        ck = s + 16 * k

        @pl.when(ck < 391)
        def _():
            pltpu.sync_copy(zero_v, acc.at[pl.ds(ck * 128, 128)])

    plsc.subcore_barrier()

    @pl.loop(0, 49)
    def _outer(i):
        rb = wid * 196 + i * 4
        pltpu.sync_copy(colp_hbm.at[pl.ds(rb, 4)], colbuf)
        for r in range(4):
            pltpu.sync_copy(ones_v, acc.at[colbuf.at[r]], add=True)

    plsc.subcore_barrier()

    pltpu.sync_copy(acc.at[pl.ds(s * 3128, 3128)], stage_v)
    pltpu.sync_copy(stage_v, out_hbm.at[pl.ds(c * NP + s * 3128, 3128)])


def _sc_agg_body(rowp_hbm, colp_hbm, hp_hbm, out_hbm, acc,
                 rowA0, colA0, rowA1, colA1, rowB0, colB0, rowB1, colB1,
                 rowsA, rowsB, zero_v,
                 siA, siB, sgA, sgB, ssA, ssB):
    c = lax.axis_index("c")
    s = lax.axis_index("s")
    rowAs, colAs = [rowA0, rowA1], [colA0, colA1]
    rowBs, colBs = [rowB0, rowB1], [colB0, colB1]

    @pl.loop(0, 64)
    def _fillz(r):
        zero_v[r, pl.ds(0, 16)] = jnp.zeros((16,), f32)
        zero_v[r, pl.ds(16, 16)] = jnp.zeros((16,), f32)

    @pl.loop(0, 49)
    def _zero(k):
        ck = s + 16 * k

        @pl.when(ck < 782)
        def _():
            pltpu.sync_copy(zero_v, acc.at[pl.ds(ck * 64, 64)])

    plsc.subcore_barrier()

    def fire_idx(g, rowb, colb, sem):
        rb = s * 392 + g * 2
        pltpu.async_copy(rowp_hbm.at[pl.ds(rb, 2)], rowb, sem)
        pltpu.async_copy(colp_hbm.at[pl.ds(rb, 2)], colb, sem)

    def wait_idx(rowb, colb, sem):
        pltpu.make_async_copy(rowp_hbm.at[pl.ds(0, 2)], rowb, sem).wait()
        pltpu.make_async_copy(colp_hbm.at[pl.ds(0, 2)], colb, sem).wait()

    def fire_g(rowb, rowsb, sem):
        for r in range(2):
            pltpu.async_copy(hp_hbm.at[c].at[rowb.at[r]],
                             rowsb.at[pl.ds(r * 128, 128)], sem)

    def wait_g(rowb, rowsb, sem):
        for r in range(2):
            pltpu.make_async_copy(hp_hbm.at[c].at[rowb.at[r]],
                                  rowsb.at[pl.ds(r * 128, 128)], sem).wait()

    def fire_s(colb, rowsb, sem):
        for r in range(2):
            pltpu.async_copy(rowsb.at[pl.ds(r * 128, 128)],
                             acc.at[colb.at[r]], sem, add=True)

    def wait_s(colb, rowsb, sem):
        for r in range(2):
            pltpu.make_async_copy(rowsb.at[pl.ds(r * 128, 128)],
                                  acc.at[colb.at[r]], sem).wait()

    fire_idx(0, rowAs[0], colAs[0], siA)
    fire_idx(1, rowBs[0], colBs[0], siB)

    def half_iter(i, p, q):
        rowA, colA = rowAs[p], colAs[p]
        rowB, colB = rowBs[p], colBs[p]

        @pl.when(i > 0)
        def _wsa():
            wait_s(colAs[q], rowsA, ssA)

        @pl.when(i < 97)
        def _pfa():
            fire_idx(2 * i + 2, rowAs[q], colAs[q], siA)

        @pl.when(i > 0)
        def _sb():
            wait_g(rowBs[q], rowsB, sgB)
            fire_s(colBs[q], rowsB, ssB)

        wait_idx(rowA, colA, siA)
        fire_g(rowA, rowsA, sgA)

        @pl.when(i > 0)
        def _wsb():
            wait_s(colBs[q], rowsB, ssB)

        @pl.when(i < 97)
        def _pfb():
            fire_idx(2 * i + 3, rowBs[q], colBs[q], siB)

        wait_idx(rowB, colB, siB)
        fire_g(rowB, rowsB, sgB)

        wait_g(rowA, rowsA, sgA)
        fire_s(colA, rowsA, ssA)

    @pl.loop(0, 49)
    def _outer(j):
        half_iter(2 * j, 0, 1)
        half_iter(2 * j + 1, 1, 0)

    wait_g(rowBs[1], rowsB, sgB)
    fire_s(colBs[1], rowsB, ssB)
    wait_s(colAs[1], rowsA, ssA)
    wait_s(colBs[1], rowsB, ssB)

    plsc.subcore_barrier()

    @pl.loop(0, 17)
    def _out(k):
        base = s * 3128 + k * 184
        pltpu.sync_copy(acc.at[pl.ds(base, 184)],
                        rowsA.at[pl.ds(0, 184)])
        pltpu.sync_copy(rowsA.at[pl.ds(0, 184)],
                        out_hbm.at[c, pl.ds(base, 184)])


_SC_MESH = plsc.VectorSubcoreMesh(core_axis_name="c", subcore_axis_name="s")


def _build_sc_deg():
    return pl.kernel(
        _sc_deg_body,
        out_type=jax.ShapeDtypeStruct((2 * NP,), f32),
        mesh=_SC_MESH,
        scratch_types=[
            pltpu.VMEM_SHARED((ACC_N,), f32),
            pltpu.VMEM((4, 128), jnp.int32),
            pltpu.VMEM((128,), f32),
            pltpu.VMEM((128,), f32),
            pltpu.VMEM((3128,), f32),
            pltpu.SemaphoreType.DMA,
        ],
    )


def _build_sc_agg():
    return pl.kernel(
        _sc_agg_body,
        out_type=jax.ShapeDtypeStruct((2, NP, HALF), f32),
        mesh=_SC_MESH,
        scratch_types=(
            [pltpu.VMEM_SHARED((ACC_N, HALF), f32)]
            + [pltpu.VMEM((2, 128), jnp.int32) for _ in range(8)]
            + [pltpu.VMEM((256, HALF), f32) for _ in range(2)]
            + [pltpu.VMEM((64, HALF), f32)]
            + [pltpu.SemaphoreType.DMA for _ in range(6)]
        ),
        compiler_params=pltpu.CompilerParams(use_tc_tiling_on_sc=False),
    )


_BLK = 1000
_NBLK = N // _BLK


def _tc1_body(x_ref, w_ref, degp_ref, hp_ref, dinv_ref):
    h = jnp.dot(x_ref[...], w_ref[...], preferred_element_type=f32)
    dtot = degp_ref[0] + degp_ref[1] + 1.0
    dinv = lax.rsqrt(dtot)
    dinv_ref[...] = dinv
    hp = h * dinv
    hp_ref[0] = hp[:, :HALF]
    hp_ref[1] = hp[:, HALF:]


def _tc_mid_body(agg_ref, hpp_ref, dinv_ref, b_ref, w_ref, hp_ref):
    a = jnp.concatenate([agg_ref[0], agg_ref[1]], axis=1)
    hpv = jnp.concatenate([hpp_ref[0], hpp_ref[1]], axis=1)
    s = jax.nn.relu(dinv_ref[...] * (a + hpv) + b_ref[...])
    h = jnp.dot(s, w_ref[...], preferred_element_type=f32)
    hp = h * dinv_ref[...]
    hp_ref[0] = hp[:, :HALF]
    hp_ref[1] = hp[:, HALF:]


def _tc_final_body(agg_ref, hpp_ref, dinv_ref, b_ref, batch_ref, wc_ref,
                   bc_ref, out_ref, psum, cnt):
    i = pl.program_id(0)
    a = jnp.concatenate([agg_ref[0], agg_ref[1]], axis=1)
    hpv = jnp.concatenate([hpp_ref[0], hpp_ref[1]], axis=1)
    h = jax.nn.relu(dinv_ref[...] * (a + hpv) + b_ref[...])
    onehot_t = (lax.broadcasted_iota(jnp.int32, (NG, _BLK), 0)
                == batch_ref[0]).astype(f32)
    ps = jnp.dot(onehot_t, h, preferred_element_type=f32)
    ct = jnp.sum(onehot_t, axis=1, keepdims=True)

    @pl.when(i == 0)
    def _init():
        psum[...] = ps
        cnt[...] = ct

    @pl.when(i > 0)
    def _acc():
        psum[...] += ps
        cnt[...] += ct

    @pl.when(i == _NBLK - 1)
    def _fin():
        pooled = psum[...] / jnp.maximum(cnt[...], 1.0)
        out_ref[...] = (jnp.dot(pooled, wc_ref[...],
                                preferred_element_type=f32) + bc_ref[...])


def _make_tc1():
    return pl.pallas_call(
        _tc1_body,
        grid=(_NBLK,),
        in_specs=[
            pl.BlockSpec((_BLK, IN_DIM), lambda i: (i, 0)),
            pl.BlockSpec((IN_DIM, HID), lambda i: (0, 0)),
            pl.BlockSpec((2, _BLK, 1), lambda i: (0, i, 0)),
        ],
        out_specs=[
            pl.BlockSpec((2, _BLK, HALF), lambda i: (0, i, 0)),
            pl.BlockSpec((_BLK, 1), lambda i: (i, 0)),
        ],
        out_shape=[
            jax.ShapeDtypeStruct((2, NP, HALF), f32),
            jax.ShapeDtypeStruct((N, 1), f32),
        ],
    )


def _make_tc_mid():
    return pl.pallas_call(
        _tc_mid_body,
        grid=(_NBLK,),
        in_specs=[
            pl.BlockSpec((2, _BLK, HALF), lambda i: (0, i, 0)),
            pl.BlockSpec((2, _BLK, HALF), lambda i: (0, i, 0)),
            pl.BlockSpec((_BLK, 1), lambda i: (i, 0)),
            pl.BlockSpec((1, HID), lambda i: (0, 0)),
            pl.BlockSpec((HID, HID), lambda i: (0, 0)),
        ],
        out_specs=[
            pl.BlockSpec((2, _BLK, HALF), lambda i: (0, i, 0)),
        ],
        out_shape=[jax.ShapeDtypeStruct((2, NP, HALF), f32)],
    )


def _make_tc_final():
    return pl.pallas_call(
        _tc_final_body,
        grid=(_NBLK,),
        in_specs=[
            pl.BlockSpec((2, _BLK, HALF), lambda i: (0, i, 0)),
            pl.BlockSpec((2, _BLK, HALF), lambda i: (0, i, 0)),
            pl.BlockSpec((_BLK, 1), lambda i: (i, 0)),
            pl.BlockSpec((1, HID), lambda i: (0, 0)),
            pl.BlockSpec((1, 1, _BLK), lambda i: (i, 0, 0)),
            pl.BlockSpec((HID, NC), lambda i: (0, 0)),
            pl.BlockSpec((1, NC), lambda i: (0, 0)),
        ],
        out_specs=pl.BlockSpec((NG, NC), lambda i: (0, 0)),
        out_shape=jax.ShapeDtypeStruct((NG, NC), f32),
        scratch_shapes=[
            pltpu.VMEM((NG, HID), f32),
            pltpu.VMEM((NG, 1), f32),
        ],
    )


@jax.jit
def kernel(x, edge_index, batch, W1, b1, W2, b2, W3, b3, Wc, bc):
    sc_deg = _build_sc_deg()
    sc_agg = _build_sc_agg()
    tc1 = _make_tc1()
    tc_mid = _make_tc_mid()
    tc_final = _make_tc_final()

    pad = EP - E
    rowp = jnp.concatenate(
        [edge_index[0], jnp.zeros((pad,), jnp.int32)]).reshape(EROWS, 128)
    colp = jnp.concatenate(
        [edge_index[1], jnp.full((pad,), TRASH, jnp.int32)]).reshape(EROWS, 128)
    batch3 = batch.reshape(_NBLK, 1, _BLK)

    degp = sc_deg(colp).reshape(2, NP, 1)
    hp1, dinv = tc1(x, W1, degp)
    agg1 = sc_agg(rowp, colp, hp1)
    hp2, = tc_mid(agg1, hp1, dinv, b1.reshape(1, HID), W2)
    agg2 = sc_agg(rowp, colp, hp2)
    hp3, = tc_mid(agg2, hp2, dinv, b2.reshape(1, HID), W3)
    agg3 = sc_agg(rowp, colp, hp3)
    return tc_final(agg3, hp3, dinv, b3.reshape(1, HID), batch3,
                    Wc, bc.reshape(1, NC))
